# baseline (speedup 1.0000x reference)
import jax
import jax.numpy as jnp
from jax.experimental import pallas as pl
from jax.experimental.pallas import tpu as pltpu


def _void_fill_body(pix_ref, tex_ref, out_ref):
    keepw = ~jax.lax.shift_right_arithmetic(pix_ref[...], 31)
    keep4 = pltpu.bitcast(keepw, jnp.int8).astype(jnp.int32)
    tex_i = pltpu.bitcast(tex_ref[...], jnp.int32)
    out_ref[...] = pltpu.bitcast(tex_i & keep4, jnp.float32)


def kernel(texels, pix_to_face):
    B, H, W, K, C = texels.shape
    L = 128
    T = W // L
    tex2 = (texels.reshape(B, H, T, L, K, C)
            .transpose(0, 1, 2, 4, 5, 3)
            .reshape(B * H * T * K * C, L))
    pix2 = pix_to_face.reshape(B * H * K * T, L)
    G = 1024
    rows = tex2.shape[0]
    out2 = pl.pallas_call(
        _void_fill_body,
        grid=(rows // (4 * C * G),),
        in_specs=[
            pl.BlockSpec((C * G, L), lambda i: (i, 0)),
            pl.BlockSpec((4 * C * G, L), lambda i: (i, 0)),
        ],
        out_specs=pl.BlockSpec((4 * C * G, L), lambda i: (i, 0)),
        out_shape=jax.ShapeDtypeStruct((rows, L), texels.dtype),
    )(pix2, tex2)
    return (out2.reshape(B, H, T, K, C, L)
            .transpose(0, 1, 2, 5, 3, 4)
            .reshape(B, H, W, K, C))
